# baseline (speedup 1.0000x reference)
import functools

import jax
import jax.numpy as jnp
from jax import lax
from jax.experimental import pallas as pl
from jax.experimental.pallas import tpu as pltpu
from jax.experimental.pallas import tpu_sc as plsc

NEG = -1.0e30
NC = 2
NSUB = 16
LANES = 16
CHUNK = 128
GRP = 16
NB = 2000
EB = 2048


def _mesh():
  return plsc.VectorSubcoreMesh(
      core_axis_name="c", subcore_axis_name="s",
      num_cores=NC, num_subcores=NSUB)


def _embed_body(x_ref, Wn_ref, bn_ref, Wa1_ref, h_ref, p_ref):
  hh = Wa1_ref.shape[1] // 2
  h = jnp.maximum(
      jnp.dot(x_ref[...], Wn_ref[...], preferred_element_type=jnp.float32)
      + bn_ref[...], 0.0)
  h_ref[...] = h
  p = jnp.dot(h, Wa1_ref[...], preferred_element_type=jnp.float32)
  p_ref[0] = p[:, :hh]
  p_ref[1] = p[:, hh:]


def _make_ea_body(E, E4, L):
  def body(eattr_ref, We_ref, be_ref, Wa2_ref, ba_ref, out_ref):
    i = pl.program_id(0)
    nb = eattr_ref.shape[1]
    hh = Wa2_ref.shape[2] // 2
    eas = []
    valids = []
    for s in range(4):
      eas.append(jnp.maximum(
          jnp.dot(eattr_ref[s], We_ref[...],
                  preferred_element_type=jnp.float32) + be_ref[...], 0.0))
      ids = s * E4 + i * nb + lax.broadcasted_iota(jnp.int32, (nb, 1), 0)
      valids.append(ids < E)
    for l in range(L):
      qs = []
      for s in range(4):
        q = (jnp.dot(eas[s], Wa2_ref[l], preferred_element_type=jnp.float32)
             + ba_ref[l][None, :])
        qs.append(jnp.where(valids[s], q, NEG))
      for c in range(2):
        out_ref[l, c] = jnp.concatenate(
            [q[:, c * hh:(c + 1) * hh] for q in qs], axis=1)
  return body


def _make_update_body(with_p):
  def body(*refs):
    if with_p:
      (agg_ref, h_ref, deg_ref, Wb_ref, bb_ref, Ws_ref, bs_ref, Wa1_ref,
       h2_ref, p_ref) = refs
    else:
      (agg_ref, h_ref, deg_ref, Wb_ref, bb_ref, Ws_ref, bs_ref,
       h2_ref) = refs
    t = jnp.concatenate([agg_ref[0], agg_ref[1]], axis=1)
    deg = deg_ref[0, :, 0] + deg_ref[1, :, 0]
    val = (jnp.dot(t, Wb_ref[...], preferred_element_type=jnp.float32)
           + deg[:, None] * bb_ref[...]
           + jnp.dot(h_ref[...], Ws_ref[...], preferred_element_type=jnp.float32)
           + bs_ref[...])
    h2 = jnp.maximum(val, 0.0)
    h2_ref[...] = h2
    if with_p:
      hh = Wa1_ref.shape[1] // 2
      p = jnp.dot(h2, Wa1_ref[...], preferred_element_type=jnp.float32)
      p_ref[0] = p[:, :hh]
      p_ref[1] = p[:, hh:]
  return body


def _make_readout_body(nblk, ng):
  def body(h_ref, batch_ref, Wr1_ref, br1_ref, Wr2_ref, br2_ref, out_ref,
           g_acc, c_acc):
    i = pl.program_id(0)
    nb = h_ref.shape[0]

    @pl.when(i == 0)
    def _():
      g_acc[...] = jnp.zeros_like(g_acc)
      c_acc[...] = jnp.zeros_like(c_acc)

    b = batch_ref[0, 0, :]
    oh = (b[:, None] == lax.broadcasted_iota(jnp.int32, (nb, ng), 1)
          ).astype(jnp.float32)
    g_acc[...] += lax.dot_general(
        oh, h_ref[...], (((0,), (0,)), ((), ())),
        preferred_element_type=jnp.float32)
    c_acc[0:1, :] += jnp.sum(oh, axis=0)[None, :]

    @pl.when(i == nblk - 1)
    def _():
      gsum = g_acc[...]
      counts = jnp.maximum(c_acc[0], 1.0)
      g = jnp.concatenate([gsum, gsum / counts[:, None]], axis=1)
      y = jnp.maximum(
          jnp.dot(g, Wr1_ref[...], preferred_element_type=jnp.float32)
          + br1_ref[...], 0.0)
      out_ref[...] = (
          jnp.dot(y, Wr2_ref[...], preferred_element_type=jnp.float32)
          + br2_ref[...])
  return body


def _make_deg_kernel(N, E, n_rows):
  rows_per_core = n_rows // NC
  rows_per_tec = rows_per_core // NSUB
  grp = 8
  n_grp = rows_per_tec // grp
  npad = ((N + 128 * NSUB - 1) // (128 * NSUB)) * 128
  nout = npad * NSUB
  zlen = npad

  scratch = [
      pltpu.VMEM((grp, CHUNK), jnp.int32),
      pltpu.VMEM((CHUNK,), jnp.float32),
      pltpu.VMEM((zlen,), jnp.float32),
      pltpu.VMEM_SHARED((nout,), jnp.float32),
      pltpu.SemaphoreType.DMA,
  ]

  @functools.partial(
      pl.kernel,
      out_type=jax.ShapeDtypeStruct((NC, nout), jnp.float32),
      mesh=_mesh(), scratch_types=scratch, name="deg_scatter",
      compiler_params=pltpu.CompilerParams(use_tc_tiling_on_sc=False))
  def k(dst_hbm, deg_out, dstst, valbuf, zb, deg_sh, sem):
    core = lax.axis_index("c")
    sub = lax.axis_index("s")
    zv = jnp.zeros((LANES,), jnp.float32)

    @pl.loop(0, zlen // LANES)
    def _(j):
      zb[pl.ds(j * LANES, LANES)] = zv

    pltpu.sync_copy(zb.at[pl.ds(0, npad)], deg_sh.at[pl.ds(sub * npad, npad)])
    plsc.subcore_barrier()

    one = jnp.ones((LANES,), jnp.float32)
    for kk in range(CHUNK // LANES):
      valbuf[pl.ds(kk * LANES, LANES)] = one

    row_base = core * rows_per_core + sub * rows_per_tec

    @pl.loop(0, n_grp)
    def _(g):
      row0 = row_base + g * grp
      pltpu.sync_copy(dst_hbm.at[pl.ds(row0, grp)], dstst)
      for t in range(grp):
        pltpu.sync_copy(valbuf, deg_sh.at[dstst.at[t]], add=True)

    plsc.subcore_barrier()
    pltpu.sync_copy(deg_sh.at[pl.ds(sub * npad, npad)],
                    deg_out.at[core].at[pl.ds(sub * npad, npad)])

  return k, nout


def _make_edge_kernel(N, E_pad, L, l, hh):
  n_rows = E_pad // CHUNK
  rows_per_tec = n_rows // NSUB
  n_grp = rows_per_tec // GRP
  agg_rows_per_tec = ((N + 8 * NSUB - 1) // (8 * NSUB)) * 8
  n_agg = agg_rows_per_tec * NSUB
  zrows = 136
  n_zcopy = agg_rows_per_tec // zrows
  assert agg_rows_per_tec % zrows == 0

  scratch = [
      pltpu.VMEM((GRP, CHUNK), jnp.int32),
      pltpu.VMEM((GRP, CHUNK), jnp.int32),
      pltpu.VMEM((CHUNK, hh), jnp.float32),
      pltpu.VMEM((CHUNK, hh), jnp.float32),
      pltpu.VMEM((CHUNK // 4, 4 * hh), jnp.float32),
      pltpu.VMEM((CHUNK // 4, 4 * hh), jnp.float32),
      pltpu.VMEM((CHUNK, hh), jnp.float32),
      pltpu.VMEM((zrows, hh), jnp.float32),
      pltpu.VMEM_SHARED((n_agg, hh), jnp.float32),
      pltpu.SemaphoreType.DMA,
      pltpu.SemaphoreType.DMA,
      pltpu.SemaphoreType.DMA,
  ]

  @functools.partial(
      pl.kernel,
      out_type=jax.ShapeDtypeStruct((NC, n_agg, hh), jnp.float32),
      mesh=_mesh(), scratch_types=scratch, name=f"edge_pass_l{l}",
      compiler_params=pltpu.CompilerParams(use_tc_tiling_on_sc=False))
  def k(p_hbm, ea_hbm, src_hbm, dst_hbm, agg_hbm,
        srcst, dstst, rows0, rows1, ea0, ea1, qbuf, zb, agg_sh,
        g0, g1, ssem):
    core = lax.axis_index("c")
    sub = lax.axis_index("s")
    zv = jnp.zeros((LANES,), jnp.float32)

    @pl.loop(0, zrows)
    def _(j):
      for kk in range(hh // LANES):
        zb[j, pl.ds(kk * LANES, LANES)] = zv

    @pl.loop(0, n_zcopy)
    def _(i):
      pltpu.sync_copy(
          zb, agg_sh.at[pl.ds(sub * agg_rows_per_tec + i * zrows, zrows)])
    plsc.subcore_barrier()

    row_base = sub * rows_per_tec
    bufs = [(rows0, ea0, g0), (rows1, ea1, g1)]

    def fire(row0, t):
      rbuf, ebuf, sem = bufs[t % 2]
      dg = pltpu.async_copy(p_hbm.at[core].at[srcst.at[t]], rbuf, sem)
      de = pltpu.async_copy(
          ea_hbm.at[l, core, pl.ds((row0 + t) * (CHUNK // 4), CHUNK // 4)],
          ebuf, sem)
      return dg, de

    @pl.loop(0, n_grp)
    def _grp(g):
      row0 = row_base + g * GRP
      pltpu.sync_copy(src_hbm.at[pl.ds(row0, GRP)], srcst)
      pltpu.sync_copy(dst_hbm.at[pl.ds(row0, GRP)], dstst)
      pend = fire(row0, 0)
      sc_pend = [None]
      for t in range(GRP):
        rbuf, ebuf, _ = bufs[t % 2]
        nxt = fire(row0, t + 1) if t + 1 < GRP else None
        if sc_pend[0] is not None:
          sc_pend[0].wait()
          sc_pend[0] = None
        pend[0].wait()
        pend[1].wait()
        pend = nxt

        @pl.loop(0, CHUNK // 8)
        def _c(g):
          vals = []
          for u in range(2):
            er = g * 2 + u
            for jj in range(4):
              for kk in range(hh // LANES):
                a = rbuf[er * 4 + jj, pl.ds(kk * LANES, LANES)]
                b = ebuf[er, pl.ds(jj * hh + kk * LANES, LANES)]
                vals.append(jnp.maximum(a + b, 0.0))
          i = 0
          for u in range(2):
            er = g * 2 + u
            for jj in range(4):
              for kk in range(hh // LANES):
                qbuf[er * 4 + jj, pl.ds(kk * LANES, LANES)] = vals[i]
                i += 1

        sc_pend[0] = pltpu.async_copy(qbuf, agg_sh.at[dstst.at[t]], ssem,
                                      add=True)
      sc_pend[0].wait()

    plsc.subcore_barrier()
    pltpu.sync_copy(
        agg_sh.at[pl.ds(sub * agg_rows_per_tec, agg_rows_per_tec)],
        agg_hbm.at[core].at[pl.ds(sub * agg_rows_per_tec, agg_rows_per_tec)])

  return k


def kernel(x, edge_index, edge_attr, batch, Wn, bn, We, be, Wa, ba, Wb, bb,
           Ws, bs, Wr1, br1, Wr2, br2):
  N, node_in = x.shape
  E, edge_in = edge_attr.shape
  L, twoH, MSG = Wa.shape
  H = twoH // 2
  OUT = Wr2.shape[1]
  NG = 64
  hh = MSG // 2

  E_pad = ((E + NSUB * CHUNK * GRP - 1) // (NSUB * CHUNK * GRP)) * (
      NSUB * CHUNK * GRP)
  pad = E_pad - E
  E4 = E_pad // 4

  def permute(v):
    return v.reshape(4, E4 // 32, 32).transpose(1, 2, 0).reshape(
        E_pad // CHUNK, CHUNK)

  src2d = permute(jnp.concatenate([edge_index[0], jnp.zeros((pad,), jnp.int32)]))
  dst2d = permute(jnp.concatenate([edge_index[1], jnp.full((pad,), N, jnp.int32)]))
  eattr_pad = jnp.concatenate(
      [edge_attr, jnp.zeros((pad, edge_in), jnp.float32)], axis=0
  ).reshape(4, E4, edge_in)

  bn2 = bn.reshape(1, H)
  be2 = be.reshape(1, H)
  br12 = br1.reshape(1, H)
  br22 = br2.reshape(1, OUT)
  Wa1 = Wa[:, :H, :]
  Wa2 = Wa[:, H:, :]

  NB4 = EB // 4
  ea_all = pl.pallas_call(
      _make_ea_body(E, E4, L),
      grid=(E4 // NB4,),
      in_specs=[
          pl.BlockSpec((4, NB4, edge_in), lambda i: (0, i, 0)),
          pl.BlockSpec((edge_in, H), lambda i: (0, 0)),
          pl.BlockSpec((1, H), lambda i: (0, 0)),
          pl.BlockSpec((L, H, MSG), lambda i: (0, 0, 0)),
          pl.BlockSpec((L, MSG), lambda i: (0, 0)),
      ],
      out_specs=pl.BlockSpec((L, NC, NB4, 4 * hh), lambda i: (0, 0, i, 0)),
      out_shape=jax.ShapeDtypeStruct((L, NC, E4, 4 * hh), jnp.float32),
  )(eattr_pad, We, be2, Wa2, ba)

  h, p = pl.pallas_call(
      _embed_body,
      grid=(N // NB,),
      in_specs=[
          pl.BlockSpec((NB, node_in), lambda i: (i, 0)),
          pl.BlockSpec((node_in, H), lambda i: (0, 0)),
          pl.BlockSpec((1, H), lambda i: (0, 0)),
          pl.BlockSpec((H, MSG), lambda i: (0, 0)),
      ],
      out_specs=[
          pl.BlockSpec((NB, H), lambda i: (i, 0)),
          pl.BlockSpec((NC, NB, hh), lambda i: (0, i, 0)),
      ],
      out_shape=[
          jax.ShapeDtypeStruct((N, H), jnp.float32),
          jax.ShapeDtypeStruct((NC, N, hh), jnp.float32),
      ],
  )(x, Wn, bn2, Wa1[0])

  deg_kernel, nout = _make_deg_kernel(N, E, E_pad // CHUNK)
  deg_parts = deg_kernel(dst2d).reshape(NC, nout, 1)

  for l in range(L):
    agg = _make_edge_kernel(N, E_pad, L, l, hh)(p, ea_all, src2d, dst2d)
    with_p = l < L - 1
    in_specs = [
        pl.BlockSpec((NC, NB, hh), lambda i: (0, i, 0)),
        pl.BlockSpec((NB, H), lambda i: (i, 0)),
        pl.BlockSpec((NC, NB, 1), lambda i: (0, i, 0)),
        pl.BlockSpec((MSG, H), lambda i: (0, 0)),
        pl.BlockSpec((1, H), lambda i: (0, 0)),
        pl.BlockSpec((H, H), lambda i: (0, 0)),
        pl.BlockSpec((1, H), lambda i: (0, 0)),
    ]
    out_specs = [pl.BlockSpec((NB, H), lambda i: (i, 0))]
    out_shape = [jax.ShapeDtypeStruct((N, H), jnp.float32)]
    args = [agg, h, deg_parts, Wb[l], bb[l].reshape(1, H), Ws[l],
            bs[l].reshape(1, H)]
    if with_p:
      in_specs.append(pl.BlockSpec((H, MSG), lambda i: (0, 0)))
      out_specs.append(pl.BlockSpec((NC, NB, hh), lambda i: (0, i, 0)))
      out_shape.append(jax.ShapeDtypeStruct((NC, N, hh), jnp.float32))
      args.append(Wa1[l + 1])
    res = pl.pallas_call(
        _make_update_body(with_p),
        grid=(N // NB,),
        in_specs=in_specs,
        out_specs=out_specs,
        out_shape=out_shape,
    )(*args)
    if with_p:
      h, p = res
    else:
      h = res[0]

  nblk = N // NB
  batch3d = batch.reshape(nblk, 1, NB)
  out = pl.pallas_call(
      _make_readout_body(nblk, NG),
      grid=(nblk,),
      in_specs=[
          pl.BlockSpec((NB, H), lambda i: (i, 0)),
          pl.BlockSpec((1, 1, NB), lambda i: (i, 0, 0)),
          pl.BlockSpec((2 * H, H), lambda i: (0, 0)),
          pl.BlockSpec((1, H), lambda i: (0, 0)),
          pl.BlockSpec((H, OUT), lambda i: (0, 0)),
          pl.BlockSpec((1, OUT), lambda i: (0, 0)),
      ],
      out_specs=pl.BlockSpec((NG, OUT), lambda i: (0, 0)),
      out_shape=jax.ShapeDtypeStruct((NG, OUT), jnp.float32),
      scratch_shapes=[
          pltpu.VMEM((NG, H), jnp.float32),
          pltpu.VMEM((8, NG), jnp.float32),
      ],
  )(h, batch3d, Wr1, br12, Wr2, br22)

  return out


# speedup vs baseline: 1.0386x; 1.0386x over previous
import functools

import jax
import jax.numpy as jnp
from jax import lax
from jax.experimental import pallas as pl
from jax.experimental.pallas import tpu as pltpu
from jax.experimental.pallas import tpu_sc as plsc

NEG = -1.0e30
NC = 2
NSUB = 16
LANES = 16
CHUNK = 128
GRP = 16
NB = 2000
EB = 2048


def _mesh():
  return plsc.VectorSubcoreMesh(
      core_axis_name="c", subcore_axis_name="s",
      num_cores=NC, num_subcores=NSUB)


def _embed_body(x_ref, Wn_ref, bn_ref, Wa1_ref, h_ref, p_ref):
  hh = Wa1_ref.shape[1] // 2
  h = jnp.maximum(
      jnp.dot(x_ref[...], Wn_ref[...], preferred_element_type=jnp.float32)
      + bn_ref[...], 0.0)
  h_ref[...] = h
  p = jnp.dot(h, Wa1_ref[...], preferred_element_type=jnp.float32)
  p_ref[0] = p[:, :hh]
  p_ref[1] = p[:, hh:]


def _make_ea_body(E, E4, n_l):
  def body(ea0_ref, ea1_ref, ea2_ref, ea3_ref, We_ref, be_ref, Wa2_ref,
           ba_ref, out_ref):
    i = pl.program_id(0)
    refs = [ea0_ref, ea1_ref, ea2_ref, ea3_ref]
    nb = ea0_ref.shape[0]
    hh = Wa2_ref.shape[2] // 2
    eas = []
    valids = []
    for s in range(4):
      eas.append(jnp.maximum(
          jnp.dot(refs[s][...], We_ref[...],
                  preferred_element_type=jnp.float32) + be_ref[...], 0.0))
      ids = s * E4 + i * nb + lax.broadcasted_iota(jnp.int32, (nb, 1), 0)
      valids.append(ids < E)
    for l in range(n_l):
      qs = []
      for s in range(4):
        q = (jnp.dot(eas[s], Wa2_ref[l], preferred_element_type=jnp.float32)
             + ba_ref[l][None, :])
        qs.append(jnp.where(valids[s], q, NEG))
      for c in range(2):
        out_ref[l, c] = jnp.concatenate(
            [q[:, c * hh:(c + 1) * hh] for q in qs], axis=1)
  return body


def _make_update_body(with_p):
  def body(*refs):
    if with_p:
      (agg_ref, h_ref, deg_ref, Wb_ref, bb_ref, Ws_ref, bs_ref, Wa1_ref,
       h2_ref, p_ref) = refs
    else:
      (agg_ref, h_ref, deg_ref, Wb_ref, bb_ref, Ws_ref, bs_ref,
       h2_ref) = refs
    t = jnp.concatenate([agg_ref[0], agg_ref[1]], axis=1)
    deg = deg_ref[0, :, 0] + deg_ref[1, :, 0]
    val = (jnp.dot(t, Wb_ref[...], preferred_element_type=jnp.float32)
           + deg[:, None] * bb_ref[...]
           + jnp.dot(h_ref[...], Ws_ref[...], preferred_element_type=jnp.float32)
           + bs_ref[...])
    h2 = jnp.maximum(val, 0.0)
    h2_ref[...] = h2
    if with_p:
      hh = Wa1_ref.shape[1] // 2
      p = jnp.dot(h2, Wa1_ref[...], preferred_element_type=jnp.float32)
      p_ref[0] = p[:, :hh]
      p_ref[1] = p[:, hh:]
  return body


def _make_readout_body(nblk, ng):
  def body(h_ref, batch_ref, Wr1_ref, br1_ref, Wr2_ref, br2_ref, out_ref,
           g_acc, c_acc):
    i = pl.program_id(0)
    nb = h_ref.shape[0]

    @pl.when(i == 0)
    def _():
      g_acc[...] = jnp.zeros_like(g_acc)
      c_acc[...] = jnp.zeros_like(c_acc)

    b = batch_ref[0, 0, :]
    oh = (b[:, None] == lax.broadcasted_iota(jnp.int32, (nb, ng), 1)
          ).astype(jnp.float32)
    g_acc[...] += lax.dot_general(
        oh, h_ref[...], (((0,), (0,)), ((), ())),
        preferred_element_type=jnp.float32)
    c_acc[0:1, :] += jnp.sum(oh, axis=0)[None, :]

    @pl.when(i == nblk - 1)
    def _():
      gsum = g_acc[...]
      counts = jnp.maximum(c_acc[0], 1.0)
      g = jnp.concatenate([gsum, gsum / counts[:, None]], axis=1)
      y = jnp.maximum(
          jnp.dot(g, Wr1_ref[...], preferred_element_type=jnp.float32)
          + br1_ref[...], 0.0)
      out_ref[...] = (
          jnp.dot(y, Wr2_ref[...], preferred_element_type=jnp.float32)
          + br2_ref[...])
  return body


def _make_deg_kernel(N, E, n_rows):
  rows_per_core = n_rows // NC
  rows_per_tec = rows_per_core // NSUB
  grp = 8
  n_grp = rows_per_tec // grp
  npad = ((N + 128 * NSUB - 1) // (128 * NSUB)) * 128
  nout = npad * NSUB
  zlen = npad

  scratch = [
      pltpu.VMEM((grp, CHUNK), jnp.int32),
      pltpu.VMEM((CHUNK,), jnp.float32),
      pltpu.VMEM((zlen,), jnp.float32),
      pltpu.VMEM_SHARED((nout,), jnp.float32),
      pltpu.SemaphoreType.DMA,
  ]

  @functools.partial(
      pl.kernel,
      out_type=jax.ShapeDtypeStruct((NC, nout), jnp.float32),
      mesh=_mesh(), scratch_types=scratch, name="deg_scatter",
      compiler_params=pltpu.CompilerParams(use_tc_tiling_on_sc=False))
  def k(dst_hbm, deg_out, dstst, valbuf, zb, deg_sh, sem):
    core = lax.axis_index("c")
    sub = lax.axis_index("s")
    zv = jnp.zeros((LANES,), jnp.float32)

    @pl.loop(0, zlen // LANES)
    def _(j):
      zb[pl.ds(j * LANES, LANES)] = zv

    pltpu.sync_copy(zb.at[pl.ds(0, npad)], deg_sh.at[pl.ds(sub * npad, npad)])
    plsc.subcore_barrier()

    one = jnp.ones((LANES,), jnp.float32)
    for kk in range(CHUNK // LANES):
      valbuf[pl.ds(kk * LANES, LANES)] = one

    row_base = core * rows_per_core + sub * rows_per_tec

    @pl.loop(0, n_grp)
    def _(g):
      row0 = row_base + g * grp
      pltpu.sync_copy(dst_hbm.at[pl.ds(row0, grp)], dstst)
      for t in range(grp):
        pltpu.sync_copy(valbuf, deg_sh.at[dstst.at[t]], add=True)

    plsc.subcore_barrier()
    pltpu.sync_copy(deg_sh.at[pl.ds(sub * npad, npad)],
                    deg_out.at[core].at[pl.ds(sub * npad, npad)])

  return k, nout


def _make_edge_kernel(N, E_pad, tag, l, hh):
  n_rows = E_pad // CHUNK
  rows_per_tec = n_rows // NSUB
  n_grp = rows_per_tec // GRP
  agg_rows_per_tec = ((N + 8 * NSUB - 1) // (8 * NSUB)) * 8
  n_agg = agg_rows_per_tec * NSUB
  zrows = 136
  n_zcopy = agg_rows_per_tec // zrows
  assert agg_rows_per_tec % zrows == 0

  scratch = [
      pltpu.VMEM((GRP, CHUNK), jnp.int32),
      pltpu.VMEM((GRP, CHUNK), jnp.int32),
      pltpu.VMEM((CHUNK, hh), jnp.float32),
      pltpu.VMEM((CHUNK, hh), jnp.float32),
      pltpu.VMEM((CHUNK // 4, 4 * hh), jnp.float32),
      pltpu.VMEM((CHUNK // 4, 4 * hh), jnp.float32),
      pltpu.VMEM((CHUNK, hh), jnp.float32),
      pltpu.VMEM((zrows, hh), jnp.float32),
      pltpu.VMEM_SHARED((n_agg, hh), jnp.float32),
      pltpu.SemaphoreType.DMA,
      pltpu.SemaphoreType.DMA,
      pltpu.SemaphoreType.DMA,
  ]

  @functools.partial(
      pl.kernel,
      out_type=jax.ShapeDtypeStruct((NC, n_agg, hh), jnp.float32),
      mesh=_mesh(), scratch_types=scratch, name=f"edge_pass_l{tag}",
      compiler_params=pltpu.CompilerParams(use_tc_tiling_on_sc=False))
  def k(p_hbm, ea_hbm, src_hbm, dst_hbm, agg_hbm,
        srcst, dstst, rows0, rows1, ea0, ea1, qbuf, zb, agg_sh,
        g0, g1, ssem):
    core = lax.axis_index("c")
    sub = lax.axis_index("s")
    zv = jnp.zeros((LANES,), jnp.float32)

    @pl.loop(0, zrows)
    def _(j):
      for kk in range(hh // LANES):
        zb[j, pl.ds(kk * LANES, LANES)] = zv

    @pl.loop(0, n_zcopy)
    def _(i):
      pltpu.sync_copy(
          zb, agg_sh.at[pl.ds(sub * agg_rows_per_tec + i * zrows, zrows)])
    plsc.subcore_barrier()

    row_base = sub * rows_per_tec
    bufs = [(rows0, ea0, g0), (rows1, ea1, g1)]

    def fire(row0, t):
      rbuf, ebuf, sem = bufs[t % 2]
      dg = pltpu.async_copy(p_hbm.at[core].at[srcst.at[t]], rbuf, sem)
      de = pltpu.async_copy(
          ea_hbm.at[l, core, pl.ds((row0 + t) * (CHUNK // 4), CHUNK // 4)],
          ebuf, sem)
      return dg, de

    @pl.loop(0, n_grp)
    def _grp(g):
      row0 = row_base + g * GRP
      pltpu.sync_copy(src_hbm.at[pl.ds(row0, GRP)], srcst)
      pltpu.sync_copy(dst_hbm.at[pl.ds(row0, GRP)], dstst)
      pend = fire(row0, 0)
      sc_pend = [None]
      for t in range(GRP):
        rbuf, ebuf, _ = bufs[t % 2]
        nxt = fire(row0, t + 1) if t + 1 < GRP else None
        if sc_pend[0] is not None:
          sc_pend[0].wait()
          sc_pend[0] = None
        pend[0].wait()
        pend[1].wait()
        pend = nxt

        @pl.loop(0, CHUNK // 8)
        def _c(g):
          vals = []
          for u in range(2):
            er = g * 2 + u
            for jj in range(4):
              for kk in range(hh // LANES):
                a = rbuf[er * 4 + jj, pl.ds(kk * LANES, LANES)]
                b = ebuf[er, pl.ds(jj * hh + kk * LANES, LANES)]
                vals.append(jnp.maximum(a + b, 0.0))
          i = 0
          for u in range(2):
            er = g * 2 + u
            for jj in range(4):
              for kk in range(hh // LANES):
                qbuf[er * 4 + jj, pl.ds(kk * LANES, LANES)] = vals[i]
                i += 1

        sc_pend[0] = pltpu.async_copy(qbuf, agg_sh.at[dstst.at[t]], ssem,
                                      add=True)
      sc_pend[0].wait()

    plsc.subcore_barrier()
    pltpu.sync_copy(
        agg_sh.at[pl.ds(sub * agg_rows_per_tec, agg_rows_per_tec)],
        agg_hbm.at[core].at[pl.ds(sub * agg_rows_per_tec, agg_rows_per_tec)])

  return k


def kernel(x, edge_index, edge_attr, batch, Wn, bn, We, be, Wa, ba, Wb, bb,
           Ws, bs, Wr1, br1, Wr2, br2):
  N, node_in = x.shape
  E, edge_in = edge_attr.shape
  L, twoH, MSG = Wa.shape
  H = twoH // 2
  OUT = Wr2.shape[1]
  NG = 64
  hh = MSG // 2

  E_pad = ((E + NSUB * CHUNK * GRP - 1) // (NSUB * CHUNK * GRP)) * (
      NSUB * CHUNK * GRP)
  pad = E_pad - E
  E4 = E_pad // 4

  def permute(v):
    return v.reshape(4, E4 // 32, 32).transpose(1, 2, 0).reshape(
        E_pad // CHUNK, CHUNK)

  src2d = permute(jnp.concatenate([edge_index[0], jnp.zeros((pad,), jnp.int32)]))
  dst2d = permute(jnp.concatenate([edge_index[1], jnp.full((pad,), N, jnp.int32)]))

  bn2 = bn.reshape(1, H)
  be2 = be.reshape(1, H)
  br12 = br1.reshape(1, H)
  br22 = br2.reshape(1, OUT)
  Wa1 = Wa[:, :H, :]
  Wa2 = Wa[:, H:, :]

  NB4 = EB // 4
  last_blk = pl.cdiv(E, NB4) - 1
  quarter_blks = E4 // NB4

  def quarter_spec(s):
    return pl.BlockSpec(
        (NB4, edge_in),
        lambda i, s=s: (jnp.minimum(s * quarter_blks + i, last_blk), 0))

  def run_ea(layers):
    n_l = len(layers)
    return pl.pallas_call(
        _make_ea_body(E, E4, n_l),
        grid=(quarter_blks,),
        in_specs=[
            quarter_spec(0), quarter_spec(1), quarter_spec(2), quarter_spec(3),
            pl.BlockSpec((edge_in, H), lambda i: (0, 0)),
            pl.BlockSpec((1, H), lambda i: (0, 0)),
            pl.BlockSpec((n_l, H, MSG), lambda i: (0, 0, 0)),
            pl.BlockSpec((n_l, MSG), lambda i: (0, 0)),
        ],
        out_specs=pl.BlockSpec((n_l, NC, NB4, 4 * hh), lambda i: (0, 0, i, 0)),
        out_shape=jax.ShapeDtypeStruct((n_l, NC, E4, 4 * hh), jnp.float32),
    )(edge_attr, edge_attr, edge_attr, edge_attr, We, be2,
      Wa2[layers[0]:layers[-1] + 1], ba[layers[0]:layers[-1] + 1])

  ea_first = run_ea([0])
  ea_rest = run_ea(list(range(1, L))) if L > 1 else None

  h, p = pl.pallas_call(
      _embed_body,
      grid=(N // NB,),
      in_specs=[
          pl.BlockSpec((NB, node_in), lambda i: (i, 0)),
          pl.BlockSpec((node_in, H), lambda i: (0, 0)),
          pl.BlockSpec((1, H), lambda i: (0, 0)),
          pl.BlockSpec((H, MSG), lambda i: (0, 0)),
      ],
      out_specs=[
          pl.BlockSpec((NB, H), lambda i: (i, 0)),
          pl.BlockSpec((NC, NB, hh), lambda i: (0, i, 0)),
      ],
      out_shape=[
          jax.ShapeDtypeStruct((N, H), jnp.float32),
          jax.ShapeDtypeStruct((NC, N, hh), jnp.float32),
      ],
  )(x, Wn, bn2, Wa1[0])

  deg_kernel, nout = _make_deg_kernel(N, E, E_pad // CHUNK)
  deg_parts = deg_kernel(dst2d).reshape(NC, nout, 1)

  for l in range(L):
    ea_buf, l_idx = (ea_first, 0) if l == 0 else (ea_rest, l - 1)
    agg = _make_edge_kernel(N, E_pad, l, l_idx, hh)(p, ea_buf, src2d, dst2d)
    with_p = l < L - 1
    in_specs = [
        pl.BlockSpec((NC, NB, hh), lambda i: (0, i, 0)),
        pl.BlockSpec((NB, H), lambda i: (i, 0)),
        pl.BlockSpec((NC, NB, 1), lambda i: (0, i, 0)),
        pl.BlockSpec((MSG, H), lambda i: (0, 0)),
        pl.BlockSpec((1, H), lambda i: (0, 0)),
        pl.BlockSpec((H, H), lambda i: (0, 0)),
        pl.BlockSpec((1, H), lambda i: (0, 0)),
    ]
    out_specs = [pl.BlockSpec((NB, H), lambda i: (i, 0))]
    out_shape = [jax.ShapeDtypeStruct((N, H), jnp.float32)]
    args = [agg, h, deg_parts, Wb[l], bb[l].reshape(1, H), Ws[l],
            bs[l].reshape(1, H)]
    if with_p:
      in_specs.append(pl.BlockSpec((H, MSG), lambda i: (0, 0)))
      out_specs.append(pl.BlockSpec((NC, NB, hh), lambda i: (0, i, 0)))
      out_shape.append(jax.ShapeDtypeStruct((NC, N, hh), jnp.float32))
      args.append(Wa1[l + 1])
    res = pl.pallas_call(
        _make_update_body(with_p),
        grid=(N // NB,),
        in_specs=in_specs,
        out_specs=out_specs,
        out_shape=out_shape,
    )(*args)
    if with_p:
      h, p = res
    else:
      h = res[0]

  nblk = N // NB
  batch3d = batch.reshape(nblk, 1, NB)
  out = pl.pallas_call(
      _make_readout_body(nblk, NG),
      grid=(nblk,),
      in_specs=[
          pl.BlockSpec((NB, H), lambda i: (i, 0)),
          pl.BlockSpec((1, 1, NB), lambda i: (i, 0, 0)),
          pl.BlockSpec((2 * H, H), lambda i: (0, 0)),
          pl.BlockSpec((1, H), lambda i: (0, 0)),
          pl.BlockSpec((H, OUT), lambda i: (0, 0)),
          pl.BlockSpec((1, OUT), lambda i: (0, 0)),
      ],
      out_specs=pl.BlockSpec((NG, OUT), lambda i: (0, 0)),
      out_shape=jax.ShapeDtypeStruct((NG, OUT), jnp.float32),
      scratch_shapes=[
          pltpu.VMEM((NG, H), jnp.float32),
          pltpu.VMEM((8, NG), jnp.float32),
      ],
  )(h, batch3d, Wr1, br12, Wr2, br22)

  return out
